# Initial kernel scaffold; baseline (speedup 1.0000x reference)
import jax
import jax.numpy as jnp
from jax.experimental import pallas as pl


def kernel(x, linear_w, bias, ffm_w):
    raise NotImplementedError("write your pallas kernel here")


# speedup vs baseline: 6.8311x; 6.8311x over previous
import functools

import jax
import jax.numpy as jnp
import numpy as np
from jax import lax
from jax.experimental import pallas as pl
from jax.experimental.pallas import tpu as pltpu
from jax.experimental.pallas import tpu_sc as plsc

NF = 26
ED = 16
TOT = 104000
B = 4096
NW = 32
GROUP = 4
ENT = 672
GROWS = GROUP * ENT
GCHUNKS = GROWS // 128
NGROUPS = B // GROUP
GPW = NGROUPS // NW
BPW = GPW // 4
PAIRS = 325

_OFFSETS = np.arange(NF, dtype=np.int32) * 4000
_II, _JJ = np.triu_indices(NF, 1)
_II = _II.astype(np.int32)
_JJ = _JJ.astype(np.int32)


def _sc_body(fidx_hbm, lidx_hbm, bias_hbm, linp_hbm, ffm_hbm, out_hbm,
             fidx_v, gbuf, lidx_v, lbuf, res_v, bias_v, sem, lsem):
    wid = lax.axis_index("s") * 2 + lax.axis_index("c")
    pltpu.sync_copy(bias_hbm, bias_v)
    bvec = bias_v[...]
    lanes = jnp.arange(16, dtype=jnp.int32)

    def lperm(val, perm):
        return lax.gather(
            val, perm[:, None],
            dimension_numbers=lax.GatherDimensionNumbers(
                offset_dims=(), collapsed_slice_dims=(0,),
                start_index_map=(0,)),
            slice_sizes=(1,),
            mode=lax.GatherScatterMode.PROMISE_IN_BOUNDS)

    def block_body(blk, _):
        resvec = jnp.zeros((16,), jnp.float32)
        for gs in range(4):
            g = wid * GPW + blk * 4 + gs
            pltpu.sync_copy(fidx_hbm.at[g], fidx_v)
            pltpu.sync_copy(lidx_hbm.at[g], lidx_v)
            copies = [
                pltpu.async_copy(ffm_hbm.at[fidx_v.at[c]],
                                 gbuf.at[pl.ds(c * 128, 128)], sem)
                for c in range(GCHUNKS)
            ]
            lcopy = pltpu.async_copy(linp_hbm.at[lidx_v], lbuf, lsem)
            for cp in copies:
                cp.wait()
            lcopy.wait()

            for s in range(GROUP):
                base = s * ENT

                def pair_chunk(t, a):
                    for u in range(13):
                        k = base + (t * 13 + u) * 2
                        a = a + gbuf[k] * gbuf[k + 1]
                    return a

                acc = lax.fori_loop(0, PAIRS // 13, pair_chunk,
                                    jnp.zeros((ED,), jnp.float32))
                for r in range(NF):
                    acc = acc + lbuf[s * NF + r]
                for sh in (8, 4, 2, 1):
                    acc = acc + lperm(acc, lanes ^ sh)
                resvec = resvec + jnp.where(lanes == gs * GROUP + s,
                                            acc, 0.0)
        sig = 1.0 / (1.0 + jnp.exp(-(resvec + bvec)))
        res_v[pl.ds(blk * 16, 16)] = sig
        return 0

    lax.fori_loop(0, BPW, block_body, 0)
    pltpu.sync_copy(res_v, out_hbm.at[pl.ds(wid * (B // NW), B // NW)])


@functools.partial(
    pl.kernel,
    mesh=plsc.VectorSubcoreMesh(core_axis_name="c", subcore_axis_name="s"),
    out_type=jax.ShapeDtypeStruct((B,), jnp.float32),
    compiler_params=pltpu.CompilerParams(use_tc_tiling_on_sc=False),
    scratch_types=[
        pltpu.VMEM((GCHUNKS, 128), jnp.int32),
        pltpu.VMEM((GROWS, ED), jnp.float32),
        pltpu.VMEM((GROUP * NF,), jnp.int32),
        pltpu.VMEM((GROUP * NF, ED), jnp.float32),
        pltpu.VMEM((B // NW,), jnp.float32),
        pltpu.VMEM((16,), jnp.float32),
        pltpu.SemaphoreType.DMA,
        pltpu.SemaphoreType.DMA,
    ],
)
def _ffm_sc(fidx_hbm, lidx_hbm, bias_hbm, linp_hbm, ffm_hbm, out_hbm,
            fidx_v, gbuf, lidx_v, lbuf, res_v, bias_v, sem, lsem):
    _sc_body(fidx_hbm, lidx_hbm, bias_hbm, linp_hbm, ffm_hbm, out_hbm,
             fidx_v, gbuf, lidx_v, lbuf, res_v, bias_v, sem, lsem)


def kernel(x, linear_w, bias, ffm_w):
    idx = (x.astype(jnp.int32) + jnp.asarray(_OFFSETS)[None, :])
    ii = jnp.asarray(_II)
    jj = jnp.asarray(_JJ)
    ea = idx[:, ii] + jj * TOT
    eb = idx[:, jj] + ii * TOT
    ent = jnp.stack([ea, eb], axis=2).reshape(B, 2 * PAIRS)
    ent = jnp.pad(ent, ((0, 0), (0, ENT - 2 * PAIRS)))
    fidx = ent.reshape(NGROUPS, GCHUNKS, 128)
    lidx = idx.reshape(NGROUPS, GROUP * NF)
    bias16 = jnp.broadcast_to(bias.astype(jnp.float32), (16,))
    linp = jnp.pad(linear_w.astype(jnp.float32), ((0, 0), (0, ED - 1)))
    ffm_flat = ffm_w.reshape(NF * TOT, ED)
    return _ffm_sc(fidx, lidx, bias16, linp, ffm_flat)


# speedup vs baseline: 7.6441x; 1.1190x over previous
import functools

import jax
import jax.numpy as jnp
import numpy as np
from jax import lax
from jax.experimental import pallas as pl
from jax.experimental.pallas import tpu as pltpu
from jax.experimental.pallas import tpu_sc as plsc

NF = 26
ED = 16
NT = 27
ROWF = NT * ED
TOT = 104000
B = 4096
NW = 32
GROUP = 2
GR = GROUP * NF
NG = B // GROUP
GPW = NG // NW
NIT = GPW // 2

_OFFSETS = np.arange(NF, dtype=np.int32) * 4000
_PAIRS = [(i, j) for i in range(NF) for j in range(i + 1, NF)]


def _compute_sample(gbuf, sbase):
    accs = [jnp.zeros((ED,), jnp.float32) for _ in range(4)]
    for p, (i, j) in enumerate(_PAIRS):
        va = gbuf[sbase + i, pl.ds(j * ED, ED)]
        vb = gbuf[sbase + j, pl.ds(i * ED, ED)]
        accs[p % 4] = accs[p % 4] + va * vb
    lacc = jnp.zeros((ED,), jnp.float32)
    for j in range(NF):
        lacc = lacc + gbuf[sbase + j, pl.ds(NF * ED, ED)]
    return (accs[0] + accs[1]) + (accs[2] + accs[3]) + lacc


def _sc_body(sidx_hbm, bias_hbm, wt_hbm, out_hbm,
             idx_a, idx_b, gbuf_a, gbuf_b, res_v, bias_v,
             sem_a, sem_b, isem_a, isem_b):
    wid = lax.axis_index("s") * 2 + lax.axis_index("c")
    pltpu.sync_copy(bias_hbm, bias_v)
    bvec = bias_v[...]
    lanes = jnp.arange(16, dtype=jnp.int32)

    def lperm(val, perm):
        return lax.gather(
            val, perm[:, None],
            dimension_numbers=lax.GatherDimensionNumbers(
                offset_dims=(), collapsed_slice_dims=(0,),
                start_index_map=(0,)),
            slice_sizes=(1,),
            mode=lax.GatherScatterMode.PROMISE_IN_BOUNDS)

    g0 = wid * GPW
    pltpu.sync_copy(sidx_hbm.at[g0], idx_a)
    pltpu.async_copy(wt_hbm.at[idx_a], gbuf_a, sem_a)
    pltpu.async_copy(sidx_hbm.at[g0 + 1], idx_b, isem_b)

    def it_body(t, resvec):
        for gi, (gbuf, gbuf_o, idx_o, sem_o, idx_p, isem_p, sem_w,
                 isem_w) in enumerate((
                (gbuf_a, gbuf_b, idx_b, sem_b, idx_a, isem_a, sem_a, isem_b),
                (gbuf_b, gbuf_a, idx_a, sem_a, idx_b, isem_b, sem_b, isem_a))):
            pltpu.make_async_copy(sidx_hbm.at[g0], idx_o, isem_w).wait()
            pltpu.async_copy(wt_hbm.at[idx_o], gbuf_o, sem_o)
            pltpu.async_copy(sidx_hbm.at[g0 + 2 * t + gi + 2], idx_p, isem_p)
            pltpu.make_async_copy(wt_hbm.at[pl.ds(0, GR)], gbuf, sem_w).wait()
            for s in range(GROUP):
                tot = _compute_sample(gbuf, s * NF)
                for sh in (8, 4, 2, 1):
                    tot = tot + lperm(tot, lanes ^ sh)
                lane_val = (4 * t + 2 * gi + s) & 15
                resvec = resvec + jnp.where(lanes == lane_val, tot, 0.0)
        sig = 1.0 / (1.0 + jnp.exp(-(resvec + bvec)))
        res_v[pl.ds((t // 4) * 16, 16)] = sig
        return jnp.where((t & 3) == 3, jnp.zeros((16,), jnp.float32), resvec)

    lax.fori_loop(0, NIT, it_body, jnp.zeros((16,), jnp.float32))
    pltpu.make_async_copy(wt_hbm.at[pl.ds(0, GR)], gbuf_a, sem_a).wait()
    pltpu.make_async_copy(sidx_hbm.at[g0], idx_b, isem_b).wait()
    pltpu.sync_copy(res_v, out_hbm.at[pl.ds(wid * (B // NW), B // NW)])


@functools.partial(
    pl.kernel,
    mesh=plsc.VectorSubcoreMesh(core_axis_name="c", subcore_axis_name="s"),
    out_type=jax.ShapeDtypeStruct((B,), jnp.float32),
    compiler_params=pltpu.CompilerParams(use_tc_tiling_on_sc=False),
    scratch_types=[
        pltpu.VMEM((GR,), jnp.int32),
        pltpu.VMEM((GR,), jnp.int32),
        pltpu.VMEM((GR, ROWF), jnp.float32),
        pltpu.VMEM((GR, ROWF), jnp.float32),
        pltpu.VMEM((B // NW,), jnp.float32),
        pltpu.VMEM((16,), jnp.float32),
        pltpu.SemaphoreType.DMA,
        pltpu.SemaphoreType.DMA,
        pltpu.SemaphoreType.DMA,
        pltpu.SemaphoreType.DMA,
    ],
)
def _ffm_sc(sidx_hbm, bias_hbm, wt_hbm, out_hbm,
            idx_a, idx_b, gbuf_a, gbuf_b, res_v, bias_v,
            sem_a, sem_b, isem_a, isem_b):
    _sc_body(sidx_hbm, bias_hbm, wt_hbm, out_hbm,
             idx_a, idx_b, gbuf_a, gbuf_b, res_v, bias_v,
             sem_a, sem_b, isem_a, isem_b)


def kernel(x, linear_w, bias, ffm_w):
    idx = (x.astype(jnp.int32) + jnp.asarray(_OFFSETS)[None, :])
    sidx = jnp.pad(idx.reshape(NG, GR), ((0, 4), (0, 0)))
    bias16 = jnp.broadcast_to(bias.astype(jnp.float32), (16,))
    linp = jnp.pad(linear_w.astype(jnp.float32), ((0, 0), (0, ED - 1)))
    wt = jnp.concatenate(
        [ffm_w.transpose(1, 0, 2), linp[:, None, :]], axis=1
    ).reshape(TOT, ROWF)
    return _ffm_sc(sidx, bias16, wt)
